# baseline (speedup 1.0000x reference)
import functools

import jax
import jax.numpy as jnp
from jax import lax
from jax.experimental import pallas as pl
from jax.experimental.pallas import tpu as pltpu
from jax.experimental.pallas import tpu_sc as plsc

NC, NS, L = 2, 16, 16
NW = NC * NS
B, SEQ = 4096, 200
WD, PD, OD = 64, 16, 96
B_PER_W = B // NW
NSEQ = 2
NCHUNK = B_PER_W // NSEQ
SPLITS = ((0, 96), (96, 104))


@functools.partial(
    pl.kernel,
    out_type=jax.ShapeDtypeStruct((B, SEQ, 128), jnp.float32),
    mesh=plsc.VectorSubcoreMesh(core_axis_name="c", subcore_axis_name="s"),
    scratch_types=[
        pltpu.VMEM((2, NSEQ, SEQ), jnp.int32),
        pltpu.VMEM((2, NSEQ, SEQ), jnp.int32),
        pltpu.VMEM((2, NSEQ, SEQ), jnp.int32),
        pltpu.VMEM((2, NSEQ, SEQ, WD), jnp.float32),
        pltpu.VMEM((2, NSEQ, SEQ, PD), jnp.float32),
        pltpu.VMEM((2, NSEQ, SEQ, PD), jnp.float32),
        pltpu.SemaphoreType.DMA,
        pltpu.SemaphoreType.DMA,
        pltpu.SemaphoreType.DMA,
        pltpu.SemaphoreType.DMA,
        pltpu.SemaphoreType.DMA,
        pltpu.SemaphoreType.DMA,
    ],
    compiler_params=pltpu.CompilerParams(use_tc_tiling_on_sc=False,
                                         needs_layout_passes=False),
)
def _embed_sc(words_hbm, head_hbm, tail_hbm, wt_hbm, ht_hbm, tt_hbm,
              out_hbm, widx_v, hidx_v, tidx_v, wrow_v, hrow_v, trow_v,
              gsem0, gsem1, wsem0, wsem1, isem0, isem1):
    wid = lax.axis_index("s") * NC + lax.axis_index("c")
    seq0 = wid * B_PER_W
    gsems = (gsem0, gsem1)
    wsems = (wsem0, wsem1)
    isems = (isem0, isem1)

    def gather_args(buf):
        args = []
        for i in range(NSEQ):
            for off, width in SPLITS:
                isl = pl.ds(off, width)
                args.append((wt_hbm.at[widx_v.at[buf, i, isl]],
                             wrow_v.at[buf, i, isl]))
                args.append((ht_hbm.at[hidx_v.at[buf, i, isl]],
                             hrow_v.at[buf, i, isl]))
                args.append((tt_hbm.at[tidx_v.at[buf, i, isl]],
                             trow_v.at[buf, i, isl]))
        return args

    def write_args(ci, buf):
        dst = out_hbm.at[pl.ds(seq0 + ci * NSEQ, NSEQ)]
        return ((wrow_v.at[buf], dst.at[:, :, pl.ds(0, WD)]),
                (hrow_v.at[buf], dst.at[:, :, pl.ds(WD, PD)]),
                (trow_v.at[buf], dst.at[:, :, pl.ds(WD + PD, PD)]))

    def idx_args(ci, buf):
        b0 = seq0 + ci * NSEQ
        return ((words_hbm.at[pl.ds(b0, NSEQ)], widx_v.at[buf]),
                (head_hbm.at[pl.ds(b0, NSEQ)], hidx_v.at[buf]),
                (tail_hbm.at[pl.ds(b0, NSEQ)], tidx_v.at[buf]))

    def load_and_fire(ci, buf):
        for src, dstv in idx_args(ci, buf):
            pltpu.sync_copy(src, dstv)
        for src, dstv in gather_args(buf):
            pltpu.async_copy(src, dstv, gsems[buf])

    load_and_fire(0, 0)

    def substep(ci, cur):
        nxt = 1 - cur

        @pl.when(ci + 1 < NCHUNK)
        def _():
            for src, dstv in idx_args(ci + 1, nxt):
                pltpu.async_copy(src, dstv, isems[nxt])

        for src, dstv in gather_args(cur):
            pltpu.make_async_copy(src, dstv, gsems[cur]).wait()

        @pl.when(ci + 1 < NCHUNK)
        def _():
            for src, dstv in idx_args(ci + 1, nxt):
                pltpu.make_async_copy(src, dstv, isems[nxt]).wait()
            for src, dstv in gather_args(nxt):
                pltpu.async_copy(src, dstv, gsems[nxt])

        @pl.when(ci >= 2)
        def _():
            for src, dstv in write_args(ci - 2, cur):
                pltpu.make_async_copy(src, dstv, wsems[cur]).wait()

        def fixup_body(i, _):
            for o in list(range(0, SEQ - L, L)) + [SEQ - L]:
                idxs = widx_v[cur, i, pl.ds(o, L)]
                msk = idxs == 0

                @pl.when(jnp.min(idxs) == 0)
                def _():
                    toks = o + lax.iota(jnp.int32, L)
                    bufv = jnp.full((L,), cur, jnp.int32)
                    seqv = jnp.full((L,), i, jnp.int32)
                    zf = jnp.zeros((L,), jnp.float32)
                    for col in range(WD):
                        plsc.store_scatter(
                            wrow_v,
                            [bufv, seqv, toks, jnp.full((L,), col, jnp.int32)],
                            zf, mask=msk)
            return 0

        lax.fori_loop(0, NSEQ, fixup_body, 0)

        for src, dstv in write_args(ci, cur):
            pltpu.async_copy(src, dstv, wsems[cur])

    def pair_body(p, _):
        substep(2 * p, 0)
        substep(2 * p + 1, 1)
        return 0

    lax.fori_loop(0, NCHUNK // 2, pair_body, 0)

    for ci, buf in ((NCHUNK - 2, 0), (NCHUNK - 1, 1)):
        for src, dstv in write_args(ci, buf):
            pltpu.make_async_copy(src, dstv, wsems[buf]).wait()


def kernel(words, head_pos, tail_pos, word_table, head_pos_table, tail_pos_table):
    ht = head_pos_table.at[0].set(0.0)
    tt = tail_pos_table.at[0].set(0.0)
    img = _embed_sc(words, head_pos, tail_pos, word_table, ht, tt)
    return img[:, :, :OD]
